# Initial kernel scaffold; baseline (speedup 1.0000x reference)
import jax
import jax.numpy as jnp
from jax.experimental import pallas as pl


def kernel(x, edge_index, W, b, alpha):
    raise NotImplementedError("write your pallas kernel here")


# speedup vs baseline: 24.1560x; 24.1560x over previous
import functools

import jax
import jax.numpy as jnp
from jax import lax
from jax.experimental import pallas as pl
from jax.experimental.pallas import tpu as pltpu
from jax.experimental.pallas import tpu_sc as plsc

N_NODES = 10000
D_IN = 128
N_ACC = 10016
NUM_TILES = 32
CHUNK = 128
N_CHUNKS = 79
BR = 1000


def _sc_mesh():
    return plsc.VectorSubcoreMesh(core_axis_name="c", subcore_axis_name="s")


def _deg_kernel(dst_hbm, zeros_hbm, deg_out, idx_v, ones_v, deg_sh, sem):
    cid = lax.axis_index("c")
    sid = lax.axis_index("s")
    wid = cid * 16 + sid

    @pl.when(sid == 0)
    def _():
        pltpu.sync_copy(zeros_hbm, deg_sh)

    for j in range(CHUNK // 16):
        ones_v[pl.ds(j * 16, 16)] = jnp.ones((16,), jnp.float32)
    pltpu.sync_copy(dst_hbm.at[wid], idx_v)
    plsc.subcore_barrier()

    def body(c, carry):
        pltpu.sync_copy(ones_v, deg_sh.at[idx_v.at[c]], add=True)
        return carry

    lax.fori_loop(0, N_CHUNKS, body, 0)
    plsc.subcore_barrier()

    @pl.when(sid == 0)
    def _():
        pltpu.sync_copy(deg_sh, deg_out.at[cid])


def _scatter_kernel(z_hbm, src_hbm, dst_hbm, zeros_hbm, acc_out,
                    srcv, dstv, rows, acc_sh, sem):
    cid = lax.axis_index("c")
    sid = lax.axis_index("s")
    wid = cid * 16 + sid

    @pl.when(sid == 0)
    def _():
        pltpu.sync_copy(zeros_hbm, acc_sh)

    pltpu.sync_copy(src_hbm.at[wid], srcv)
    pltpu.sync_copy(dst_hbm.at[wid], dstv)
    plsc.subcore_barrier()

    def body(c, carry):
        pltpu.async_copy(z_hbm.at[srcv.at[c]], rows, sem).wait()
        pltpu.sync_copy(rows, acc_sh.at[dstv.at[c]], add=True)
        return carry

    lax.fori_loop(0, N_CHUNKS, body, 0)
    plsc.subcore_barrier()

    @pl.when(sid == 0)
    def _():
        pltpu.sync_copy(acc_sh, acc_out.at[cid])


def _scale_kernel(x_ref, d_ref, z_ref):
    deg = d_ref[:, 0:1] + d_ref[:, 1:2] + 1.0
    z_ref[...] = x_ref[...] * lax.rsqrt(deg)


def _out_kernel(a0_ref, a1_ref, z_ref, d_ref, w_ref, b_ref, al_ref, o_ref):
    deg = d_ref[:, 0:1] + d_ref[:, 1:2] + 1.0
    s = (a0_ref[0] + a1_ref[0] + z_ref[...]) * lax.rsqrt(deg)
    o = jnp.dot(s, w_ref[...], preferred_element_type=jnp.float32)
    o = o + b_ref[...]
    o_ref[...] = jnp.where(o > 0, o, al_ref[...] * o)


def kernel(x, edge_index, W, b, alpha):
    n, d_in = x.shape
    e = edge_index.shape[1]
    d_out = W.shape[1]
    e_pad = NUM_TILES * N_CHUNKS * CHUNK

    src = jnp.concatenate(
        [edge_index[0], jnp.zeros((e_pad - e,), jnp.int32)]
    ).reshape(NUM_TILES, N_CHUNKS, CHUNK)
    dst = jnp.concatenate(
        [edge_index[1], jnp.full((e_pad - e,), n, jnp.int32)]
    ).reshape(NUM_TILES, N_CHUNKS, CHUNK)

    zeros_deg = jnp.zeros((N_ACC,), jnp.float32)
    zeros_acc = jnp.zeros((N_ACC, D_IN), jnp.float32)

    deg_fn = pl.kernel(
        _deg_kernel,
        out_type=jax.ShapeDtypeStruct((2, N_ACC), jnp.float32),
        mesh=_sc_mesh(),
        scratch_types=[
            pltpu.VMEM((N_CHUNKS, CHUNK), jnp.int32),
            pltpu.VMEM((CHUNK,), jnp.float32),
            pltpu.VMEM_SHARED((N_ACC,), jnp.float32),
            pltpu.SemaphoreType.DMA,
        ],
    )
    deg2 = deg_fn(dst, zeros_deg)

    deg_pair = deg2[:, :n].T

    grid = n // BR
    z = pl.pallas_call(
        _scale_kernel,
        grid=(grid,),
        in_specs=[
            pl.BlockSpec((BR, d_in), lambda i: (i, 0)),
            pl.BlockSpec((BR, 2), lambda i: (i, 0)),
        ],
        out_specs=pl.BlockSpec((BR, d_in), lambda i: (i, 0)),
        out_shape=jax.ShapeDtypeStruct((n, d_in), jnp.float32),
    )(x, deg_pair)

    scatter_fn = pl.kernel(
        _scatter_kernel,
        out_type=jax.ShapeDtypeStruct((2, N_ACC, D_IN), jnp.float32),
        mesh=_sc_mesh(),
        scratch_types=[
            pltpu.VMEM((N_CHUNKS, CHUNK), jnp.int32),
            pltpu.VMEM((N_CHUNKS, CHUNK), jnp.int32),
            pltpu.VMEM((CHUNK, D_IN), jnp.float32),
            pltpu.VMEM_SHARED((N_ACC, D_IN), jnp.float32),
            pltpu.SemaphoreType.DMA,
        ],
    )
    acc = scatter_fn(z, src, dst, zeros_acc)

    out = pl.pallas_call(
        _out_kernel,
        grid=(grid,),
        in_specs=[
            pl.BlockSpec((1, BR, d_in), lambda i: (0, i, 0)),
            pl.BlockSpec((1, BR, d_in), lambda i: (1, i, 0)),
            pl.BlockSpec((BR, d_in), lambda i: (i, 0)),
            pl.BlockSpec((BR, 2), lambda i: (i, 0)),
            pl.BlockSpec((d_in, d_out), lambda i: (0, 0)),
            pl.BlockSpec((1, d_out), lambda i: (0, 0)),
            pl.BlockSpec((1, d_out), lambda i: (0, 0)),
        ],
        out_specs=pl.BlockSpec((BR, d_out), lambda i: (i, 0)),
        out_shape=jax.ShapeDtypeStruct((n, d_out), jnp.float32),
    )(acc, acc, z, deg_pair, W, b.reshape(1, d_out), alpha.reshape(1, d_out))
    return out


# speedup vs baseline: 42.8649x; 1.7745x over previous
import functools

import jax
import jax.numpy as jnp
from jax import lax
from jax.experimental import pallas as pl
from jax.experimental.pallas import tpu as pltpu
from jax.experimental.pallas import tpu_sc as plsc

N_NODES = 10000
D_IN = 128
HALF = 5000
N_ACC = 10016
N_ACC_H = 5016
NUM_TILES = 32
CHUNK = 128
NCH_DEG = 80
NCH_SC = 157
NB = 3
BR = 1000


def _sc_mesh():
    return plsc.VectorSubcoreMesh(core_axis_name="c", subcore_axis_name="s")


def _deg_kernel(dst_hbm, zeros_hbm, deg_out, idx_v, ones_v, deg_sh, sem):
    cid = lax.axis_index("c")
    sid = lax.axis_index("s")
    wid = cid * 16 + sid

    @pl.when(sid == 0)
    def _():
        pltpu.sync_copy(zeros_hbm, deg_sh)

    for j in range(CHUNK // 16):
        ones_v[pl.ds(j * 16, 16)] = jnp.ones((16,), jnp.float32)
    pltpu.sync_copy(dst_hbm.at[wid], idx_v)
    plsc.subcore_barrier()

    def body(c, carry):
        pltpu.sync_copy(ones_v, deg_sh.at[idx_v.at[c]], add=True)
        return carry

    lax.fori_loop(0, NCH_DEG, body, 0)
    plsc.subcore_barrier()

    @pl.when(sid == 0)
    def _():
        pltpu.sync_copy(deg_sh, deg_out.at[cid])


LA = 2


def _scatter_kernel(z_hbm, src_hbm, dst_hbm, zeros_hbm, acc_out,
                    srcv, dstv, bufs, acc_sh, gsem, ssem):
    cid = lax.axis_index("c")
    sid = lax.axis_index("s")
    wid = cid * 16 + sid

    @pl.when(sid == 0)
    def _():
        pltpu.sync_copy(zeros_hbm, acc_sh)

    pltpu.sync_copy(src_hbm.at[wid], srcv)
    pltpu.sync_copy(dst_hbm.at[wid], dstv)
    plsc.subcore_barrier()

    def step(s, carry):
        j = lax.rem(s, NB)

        @pl.when(jnp.logical_and(s >= NB, s - NB < NCH_SC))
        def _():
            pltpu.make_async_copy(
                bufs.at[j], acc_sh.at[dstv.at[s - NB]], ssem.at[j]).wait()

        @pl.when(s < NCH_SC)
        def _():
            pltpu.async_copy(z_hbm.at[srcv.at[s]], bufs.at[j], gsem.at[j])

        @pl.when(jnp.logical_and(s >= LA, s - LA < NCH_SC))
        def _():
            c = s - LA
            jc = lax.rem(c, NB)
            pltpu.make_async_copy(
                z_hbm.at[srcv.at[c]], bufs.at[jc], gsem.at[jc]).wait()
            pltpu.async_copy(
                bufs.at[jc], acc_sh.at[dstv.at[c]], ssem.at[jc], add=True)

        return carry

    lax.fori_loop(0, NCH_SC + NB, step, 0)
    plsc.subcore_barrier()

    @pl.when(sid == 0)
    def _():
        pltpu.sync_copy(acc_sh, acc_out.at[cid])


def _scale_kernel(x_ref, d_ref, z_ref):
    deg = d_ref[:, 0:1] + d_ref[:, 1:2] + 1.0
    z_ref[...] = x_ref[...] * lax.rsqrt(deg)


def _out_kernel(a_ref, z_ref, d_ref, w_ref, b_ref, al_ref, o_ref):
    deg = d_ref[:, 0:1] + d_ref[:, 1:2] + 1.0
    s = (a_ref[0] + z_ref[...]) * lax.rsqrt(deg)
    o = jnp.dot(s, w_ref[...], preferred_element_type=jnp.float32)
    o = o + b_ref[...]
    o_ref[...] = jnp.where(o > 0, o, al_ref[...] * o)


def kernel(x, edge_index, W, b, alpha):
    n, d_in = x.shape
    e = edge_index.shape[1]
    d_out = W.shape[1]

    e_pad_deg = NUM_TILES * NCH_DEG * CHUNK
    pad16 = jnp.arange(e_pad_deg - e, dtype=jnp.int32) % 16
    dst_deg = jnp.concatenate([edge_index[1], n + pad16]).reshape(
        NUM_TILES, NCH_DEG, CHUNK)

    e_pad_sc = 16 * NCH_SC * CHUNK
    padsc = jnp.arange(e_pad_sc - e, dtype=jnp.int32) % 16
    src_h = jnp.concatenate([edge_index[0], padsc]).reshape(
        1, 16, NCH_SC, CHUNK)
    src2 = jnp.concatenate([src_h, src_h], axis=0).reshape(
        NUM_TILES, NCH_SC, CHUNK)
    dstp = jnp.concatenate([edge_index[1], n + padsc])
    junk = HALF + (jnp.arange(dstp.shape[0], dtype=jnp.int32) % 16)
    dst_lo = jnp.where(dstp < HALF, dstp, junk)
    dst_hi = jnp.where(dstp >= HALF, dstp - HALF, junk)
    dst2 = jnp.concatenate(
        [dst_lo.reshape(1, 16, NCH_SC, CHUNK),
         dst_hi.reshape(1, 16, NCH_SC, CHUNK)], axis=0
    ).reshape(NUM_TILES, NCH_SC, CHUNK)

    zeros_deg = jnp.zeros((N_ACC,), jnp.float32)
    zeros_acc = jnp.zeros((N_ACC_H, D_IN), jnp.float32)

    deg_fn = pl.kernel(
        _deg_kernel,
        out_type=jax.ShapeDtypeStruct((2, N_ACC), jnp.float32),
        mesh=_sc_mesh(),
        scratch_types=[
            pltpu.VMEM((NCH_DEG, CHUNK), jnp.int32),
            pltpu.VMEM((CHUNK,), jnp.float32),
            pltpu.VMEM_SHARED((N_ACC,), jnp.float32),
            pltpu.SemaphoreType.DMA,
        ],
    )
    deg2 = deg_fn(dst_deg, zeros_deg)

    deg_pair = deg2[:, :n].T

    grid = n // BR
    z = pl.pallas_call(
        _scale_kernel,
        grid=(grid,),
        in_specs=[
            pl.BlockSpec((BR, d_in), lambda i: (i, 0)),
            pl.BlockSpec((BR, 2), lambda i: (i, 0)),
        ],
        out_specs=pl.BlockSpec((BR, d_in), lambda i: (i, 0)),
        out_shape=jax.ShapeDtypeStruct((n, d_in), jnp.float32),
    )(x, deg_pair)

    scatter_fn = pl.kernel(
        _scatter_kernel,
        out_type=jax.ShapeDtypeStruct((2, N_ACC_H, D_IN), jnp.float32),
        mesh=_sc_mesh(),
        scratch_types=[
            pltpu.VMEM((NCH_SC, CHUNK), jnp.int32),
            pltpu.VMEM((NCH_SC, CHUNK), jnp.int32),
            pltpu.VMEM((NB, CHUNK, D_IN), jnp.float32),
            pltpu.VMEM_SHARED((N_ACC_H, D_IN), jnp.float32),
            pltpu.SemaphoreType.DMA((NB,)),
            pltpu.SemaphoreType.DMA((NB,)),
        ],
    )
    acc = scatter_fn(z, src2, dst2, zeros_acc)

    out = pl.pallas_call(
        _out_kernel,
        grid=(grid,),
        in_specs=[
            pl.BlockSpec((1, BR, d_in), lambda i: (i // 5, i % 5, 0)),
            pl.BlockSpec((BR, d_in), lambda i: (i, 0)),
            pl.BlockSpec((BR, 2), lambda i: (i, 0)),
            pl.BlockSpec((d_in, d_out), lambda i: (0, 0)),
            pl.BlockSpec((1, d_out), lambda i: (0, 0)),
            pl.BlockSpec((1, d_out), lambda i: (0, 0)),
        ],
        out_specs=pl.BlockSpec((BR, d_out), lambda i: (i, 0)),
        out_shape=jax.ShapeDtypeStruct((n, d_out), jnp.float32),
    )(acc, z, deg_pair, W, b.reshape(1, d_out), alpha.reshape(1, d_out))
    return out
